# Initial kernel scaffold; baseline (speedup 1.0000x reference)
import functools

import jax
import jax.numpy as jnp
from jax import lax
from jax.experimental import pallas as pl
from jax.experimental.pallas import tpu as pltpu
from jax.experimental.pallas import tpu_sc as plsc

_N = 10000
_E = 160000
_F_IN = 256
_H = 512

_N_PAD = 10240
_ROW_BLK = 512
_N_TILES = _N_PAD // _ROW_BLK
_E_PAD = 163840
_STEPS = 80
_DEG_STEPS = 40
_NSC = 2
_NT = 16
_STRIPE = _N_PAD // _NT


def _fill_2d(ref, rows, value):
    vec = jnp.full((16,), value, dtype=jnp.float32)

    def body(t, _):
        r = t // 8
        q = t % 8
        ref[r, pl.ds(q * 16, 16)] = vec
        return 0

    lax.fori_loop(0, rows * 8, body, 0)


def _deg_sc(dst3):
    mesh = plsc.VectorSubcoreMesh(core_axis_name="c", subcore_axis_name="s")

    @functools.partial(
        pl.kernel,
        out_type=jax.ShapeDtypeStruct((_NSC * _N_PAD, 128), jnp.float32),
        mesh=mesh,
        scratch_types=[
            pltpu.VMEM((_DEG_STEPS, 128), jnp.int32),
            pltpu.VMEM((128, 128), jnp.float32),
            pltpu.VMEM((128, 128), jnp.float32),
            pltpu.VMEM_SHARED((_N_PAD, 128), jnp.float32),
        ],
    )
    def body(dst_hbm, out_hbm, dst_v, ones_v, zer_v, acc_sh):
        cid = lax.axis_index("c")
        sid = lax.axis_index("s")
        wid = cid * _NT + sid
        pltpu.sync_copy(dst_hbm.at[wid], dst_v)
        _fill_2d(ones_v, 128, 1.0)
        _fill_2d(zer_v, 128, 0.0)
        for k in range(_STRIPE // 128):
            pltpu.sync_copy(zer_v, acc_sh.at[pl.ds(sid * _STRIPE + k * 128, 128)])
        plsc.subcore_barrier()

        def step(j, _):
            pltpu.sync_copy(ones_v, acc_sh.at[dst_v.at[j]], add=True)
            return 0

        lax.fori_loop(0, _DEG_STEPS, step, 0)
        plsc.subcore_barrier()
        pltpu.sync_copy(
            acc_sh.at[pl.ds(sid * _STRIPE, _STRIPE)],
            out_hbm.at[pl.ds(cid * _N_PAD + sid * _STRIPE, _STRIPE)],
        )

    return body(dst3)


def _scatter_sc(hp_flat, src3, dst3):
    mesh = plsc.VectorSubcoreMesh(core_axis_name="c", subcore_axis_name="s")

    @functools.partial(
        pl.kernel,
        out_type=jax.ShapeDtypeStruct((4 * _N_PAD, 128), jnp.float32),
        mesh=mesh,
        scratch_types=[
            pltpu.VMEM((_STEPS, 128), jnp.int32),
            pltpu.VMEM((_STEPS, 128), jnp.int32),
            pltpu.VMEM((_STEPS, 128), jnp.int32),
            pltpu.VMEM((128, 128), jnp.float32),
            pltpu.VMEM((128, 128), jnp.float32),
            pltpu.VMEM_SHARED((_N_PAD, 128), jnp.float32),
            pltpu.SemaphoreType.DMA,
        ],
    )
    def body(hp_hbm, src_hbm, dst_hbm, out_hbm,
             src_v, srco_v, dst_v, rows_v, zer_v, acc_sh, sem):
        cid = lax.axis_index("c")
        sid = lax.axis_index("s")
        pltpu.sync_copy(src_hbm.at[sid], src_v)
        pltpu.sync_copy(dst_hbm.at[sid], dst_v)
        _fill_2d(zer_v, 128, 0.0)
        for k in range(_STRIPE // 128):
            pltpu.sync_copy(zer_v, acc_sh.at[pl.ds(sid * _STRIPE + k * 128, 128)])

        for cc in range(2):
            chunk = cid * 2 + cc
            off = chunk * _N_PAD

            def add_off(t, _, off=off):
                r = t // 8
                q = t % 8
                offv = jnp.full((16,), off, dtype=jnp.int32)
                srco_v[r, pl.ds(q * 16, 16)] = src_v[r, pl.ds(q * 16, 16)] + offv
                return 0

            lax.fori_loop(0, _STEPS * 8, add_off, 0)
            plsc.subcore_barrier()

            def step(j, _):
                pltpu.async_copy(hp_hbm.at[srco_v.at[j]], rows_v, sem).wait()
                pltpu.sync_copy(rows_v, acc_sh.at[dst_v.at[j]], add=True)
                return 0

            lax.fori_loop(0, _STEPS, step, 0)
            plsc.subcore_barrier()
            pltpu.sync_copy(
                acc_sh.at[pl.ds(sid * _STRIPE, _STRIPE)],
                out_hbm.at[pl.ds(off + sid * _STRIPE, _STRIPE)],
            )
            if cc == 0:
                for k in range(_STRIPE // 128):
                    pltpu.sync_copy(
                        zer_v, acc_sh.at[pl.ds(sid * _STRIPE + k * 128, 128)])

    return body(hp_flat, src3, dst3)


def _mm_first_body(x_ref, degf_ref, w_ref, hp_ref, dinv_ref):
    deg = degf_ref[0, :, 0:1] + degf_ref[1, :, 0:1] + 1.0
    dinv = lax.rsqrt(deg)
    h = jnp.dot(x_ref[...], w_ref[...], preferred_element_type=jnp.float32)
    hp = h * dinv
    for c in range(4):
        hp_ref[c] = hp[:, c * 128:(c + 1) * 128]
    dinv_ref[...] = dinv


def _mm_first(x_pad, degf, w1):
    return pl.pallas_call(
        _mm_first_body,
        grid=(_N_TILES,),
        in_specs=[
            pl.BlockSpec((_ROW_BLK, _F_IN), lambda i: (i, 0)),
            pl.BlockSpec((2, _ROW_BLK, 128), lambda i: (0, i, 0)),
            pl.BlockSpec((_F_IN, _H), lambda i: (0, 0)),
        ],
        out_specs=[
            pl.BlockSpec((4, _ROW_BLK, 128), lambda i: (0, i, 0)),
            pl.BlockSpec((_ROW_BLK, 1), lambda i: (i, 0)),
        ],
        out_shape=[
            jax.ShapeDtypeStruct((4, _N_PAD, 128), jnp.float32),
            jax.ShapeDtypeStruct((_N_PAD, 1), jnp.float32),
        ],
    )(x_pad, degf, w1)


def _mm_mid_body(agg_ref, hp_ref, dinv_ref, b_ref, w_ref, out_ref):
    dinv = dinv_ref[...]
    acc = jnp.zeros((_ROW_BLK, _H), dtype=jnp.float32)
    for c in range(4):
        zc = dinv * (agg_ref[c] + hp_ref[c]) + b_ref[0, c * 128:(c + 1) * 128]
        acc = acc + jnp.dot(zc, w_ref[pl.ds(c * 128, 128), :],
                            preferred_element_type=jnp.float32)
    hpn = acc * dinv
    for c in range(4):
        out_ref[c] = hpn[:, c * 128:(c + 1) * 128]


def _mm_mid(agg, hp, dinv, b, w):
    return pl.pallas_call(
        _mm_mid_body,
        grid=(_N_TILES,),
        in_specs=[
            pl.BlockSpec((4, _ROW_BLK, 128), lambda i: (0, i, 0)),
            pl.BlockSpec((4, _ROW_BLK, 128), lambda i: (0, i, 0)),
            pl.BlockSpec((_ROW_BLK, 1), lambda i: (i, 0)),
            pl.BlockSpec((1, _H), lambda i: (0, 0)),
            pl.BlockSpec((_H, _H), lambda i: (0, 0)),
        ],
        out_specs=pl.BlockSpec((4, _ROW_BLK, 128), lambda i: (0, i, 0)),
        out_shape=jax.ShapeDtypeStruct((4, _N_PAD, 128), jnp.float32),
    )(agg, hp, dinv, b, w)


def _final_body(agg_ref, hp_ref, dinv_ref, b_ref, out_ref):
    dinv = dinv_ref[...]
    for c in range(4):
        out_ref[:, pl.ds(c * 128, 128)] = (
            dinv * (agg_ref[c] + hp_ref[c]) + b_ref[0, c * 128:(c + 1) * 128])


def _final(agg, hp, dinv, b):
    return pl.pallas_call(
        _final_body,
        grid=(_N_TILES,),
        in_specs=[
            pl.BlockSpec((4, _ROW_BLK, 128), lambda i: (0, i, 0)),
            pl.BlockSpec((4, _ROW_BLK, 128), lambda i: (0, i, 0)),
            pl.BlockSpec((_ROW_BLK, 1), lambda i: (i, 0)),
            pl.BlockSpec((1, _H), lambda i: (0, 0)),
        ],
        out_specs=pl.BlockSpec((_ROW_BLK, _H), lambda i: (i, 0)),
        out_shape=jax.ShapeDtypeStruct((_N_PAD, _H), jnp.float32),
    )(agg, hp, dinv, b)


def kernel(x, edge_index, W1, b1, W2, b2, W3, b3):
    npad = _E_PAD - _E
    src_p = jnp.concatenate(
        [edge_index[0], jnp.zeros((npad,), dtype=jnp.int32)])
    dst_p = jnp.concatenate(
        [edge_index[1], jnp.full((npad,), _N, dtype=jnp.int32)])
    src3 = src_p.reshape(_NT, _STEPS, 128)
    dst3 = dst_p.reshape(_NT, _STEPS, 128)
    dst3_deg = dst_p.reshape(_NSC * _NT, _DEG_STEPS, 128)
    x_pad = jnp.pad(x, ((0, _N_PAD - _N), (0, 0)))

    degf = _deg_sc(dst3_deg).reshape(_NSC, _N_PAD, 128)
    hp1, dinv = _mm_first(x_pad, degf, W1)
    agg1 = _scatter_sc(hp1.reshape(4 * _N_PAD, 128), src3, dst3)
    hp2 = _mm_mid(agg1.reshape(4, _N_PAD, 128), hp1, dinv,
                  b1.reshape(1, _H), W2)
    agg2 = _scatter_sc(hp2.reshape(4 * _N_PAD, 128), src3, dst3)
    hp3 = _mm_mid(agg2.reshape(4, _N_PAD, 128), hp2, dinv,
                  b2.reshape(1, _H), W3)
    agg3 = _scatter_sc(hp3.reshape(4 * _N_PAD, 128), src3, dst3)
    z = _final(agg3.reshape(4, _N_PAD, 128), hp3, dinv, b3.reshape(1, _H))
    return z[:_N]


# speedup vs baseline: 4.9372x; 4.9372x over previous
import functools

import jax
import jax.numpy as jnp
from jax import lax
from jax.experimental import pallas as pl
from jax.experimental.pallas import tpu as pltpu
from jax.experimental.pallas import tpu_sc as plsc

_N = 10000
_E = 160000
_F_IN = 256
_H = 512

_N_PAD = 10240
_ROW_BLK = 512
_N_TILES = _N_PAD // _ROW_BLK
_E_PAD = 163840
_STEPS = 80
_DEG_STEPS = 40
_NSC = 2
_NT = 16
_STRIPE = _N_PAD // _NT


def _fill_2d(ref, rows, value):
    vec = jnp.full((16,), value, dtype=jnp.float32)

    def body(t, _):
        r = t // 8
        q = t % 8
        ref[r, pl.ds(q * 16, 16)] = vec
        return 0

    lax.fori_loop(0, rows * 8, body, 0)


def _deg_sc(dst3):
    mesh = plsc.VectorSubcoreMesh(core_axis_name="c", subcore_axis_name="s", num_cores=_NSC, num_subcores=_NT)

    @functools.partial(
        pl.kernel,
        out_type=jax.ShapeDtypeStruct((_NSC * _N_PAD, 128), jnp.float32),
        mesh=mesh,
        scratch_types=[
            pltpu.VMEM((_DEG_STEPS, 128), jnp.int32),
            pltpu.VMEM((128, 128), jnp.float32),
            pltpu.VMEM_SHARED((_N_PAD, 128), jnp.float32),
        ],
    )
    def body(dst_hbm, out_hbm, dst_v, ones_v, acc_sh):
        cid = lax.axis_index("c")
        sid = lax.axis_index("s")
        wid = cid * _NT + sid
        pltpu.sync_copy(dst_hbm.at[wid], dst_v)
        _fill_2d(ones_v, 128, 0.0)
        for k in range(_STRIPE // 128):
            pltpu.sync_copy(ones_v, acc_sh.at[pl.ds(sid * _STRIPE + k * 128, 128)])
        _fill_2d(ones_v, 128, 1.0)
        plsc.subcore_barrier()

        def step(j, _):
            pltpu.sync_copy(ones_v, acc_sh.at[dst_v.at[j]], add=True)
            return 0

        lax.fori_loop(0, _DEG_STEPS, step, 0)
        plsc.subcore_barrier()
        pltpu.sync_copy(
            acc_sh.at[pl.ds(sid * _STRIPE, _STRIPE)],
            out_hbm.at[pl.ds(cid * _N_PAD + sid * _STRIPE, _STRIPE)],
        )

    return body(dst3)


def _scatter_sc(hp_flat, src3, dst3):
    mesh = plsc.VectorSubcoreMesh(core_axis_name="c", subcore_axis_name="s", num_cores=_NSC, num_subcores=_NT)

    @functools.partial(
        pl.kernel,
        out_type=jax.ShapeDtypeStruct((4 * _N_PAD, 128), jnp.float32),
        mesh=mesh,
        scratch_types=[
            pltpu.VMEM((_STEPS, 128), jnp.int32),
            pltpu.VMEM((_STEPS, 128), jnp.int32),
            pltpu.VMEM((128, 128), jnp.float32),
            pltpu.VMEM_SHARED((_N_PAD, 128), jnp.float32),
            pltpu.SemaphoreType.DMA,
        ],
    )
    def body(hp_hbm, src_hbm, dst_hbm, out_hbm,
             src_v, dst_v, rows_v, acc_sh, sem):
        cid = lax.axis_index("c")
        sid = lax.axis_index("s")
        pltpu.sync_copy(src_hbm.at[sid], src_v)
        pltpu.sync_copy(dst_hbm.at[sid], dst_v)
        _fill_2d(rows_v, 128, 0.0)
        for k in range(_STRIPE // 128):
            pltpu.sync_copy(rows_v, acc_sh.at[pl.ds(sid * _STRIPE + k * 128, 128)])

        def add_off(t, _, off_val=0):
            r = t // 8
            q = t % 8
            offv = jnp.full((16,), off_val, dtype=jnp.int32)
            src_v[r, pl.ds(q * 16, 16)] = src_v[r, pl.ds(q * 16, 16)] + offv
            return 0

        lax.fori_loop(0, _STEPS * 8, functools.partial(add_off, off_val=cid * 2 * _N_PAD), 0)

        for cc in range(2):
            off = (cid * 2 + cc) * _N_PAD
            plsc.subcore_barrier()

            def step(j, _):
                pltpu.async_copy(hp_hbm.at[src_v.at[j]], rows_v, sem).wait()
                pltpu.sync_copy(rows_v, acc_sh.at[dst_v.at[j]], add=True)
                return 0

            lax.fori_loop(0, _STEPS, step, 0)
            plsc.subcore_barrier()
            pltpu.sync_copy(
                acc_sh.at[pl.ds(sid * _STRIPE, _STRIPE)],
                out_hbm.at[pl.ds(off + sid * _STRIPE, _STRIPE)],
            )
            if cc == 0:
                _fill_2d(rows_v, 128, 0.0)
                for k in range(_STRIPE // 128):
                    pltpu.sync_copy(
                        rows_v, acc_sh.at[pl.ds(sid * _STRIPE + k * 128, 128)])
                lax.fori_loop(0, _STEPS * 8,
                              functools.partial(add_off, off_val=_N_PAD), 0)

    return body(hp_flat, src3, dst3)


def _mm_first_body(x_ref, degf_ref, w_ref, hp_ref, dinv_ref):
    deg = degf_ref[0, :, 0:1] + degf_ref[1, :, 0:1] + 1.0
    dinv = lax.rsqrt(deg)
    h = jnp.dot(x_ref[...], w_ref[...], preferred_element_type=jnp.float32)
    hp = h * dinv
    for c in range(4):
        hp_ref[c] = hp[:, c * 128:(c + 1) * 128]
    dinv_ref[...] = dinv


def _mm_first(x_pad, degf, w1):
    return pl.pallas_call(
        _mm_first_body,
        grid=(_N_TILES,),
        in_specs=[
            pl.BlockSpec((_ROW_BLK, _F_IN), lambda i: (i, 0)),
            pl.BlockSpec((2, _ROW_BLK, 128), lambda i: (0, i, 0)),
            pl.BlockSpec((_F_IN, _H), lambda i: (0, 0)),
        ],
        out_specs=[
            pl.BlockSpec((4, _ROW_BLK, 128), lambda i: (0, i, 0)),
            pl.BlockSpec((_ROW_BLK, 1), lambda i: (i, 0)),
        ],
        out_shape=[
            jax.ShapeDtypeStruct((4, _N_PAD, 128), jnp.float32),
            jax.ShapeDtypeStruct((_N_PAD, 1), jnp.float32),
        ],
    )(x_pad, degf, w1)


def _mm_mid_body(agg_ref, hp_ref, dinv_ref, b_ref, w_ref, out_ref):
    dinv = dinv_ref[...]
    acc = jnp.zeros((_ROW_BLK, _H), dtype=jnp.float32)
    for c in range(4):
        zc = dinv * (agg_ref[c] + hp_ref[c]) + b_ref[0, c * 128:(c + 1) * 128]
        acc = acc + jnp.dot(zc, w_ref[pl.ds(c * 128, 128), :],
                            preferred_element_type=jnp.float32)
    hpn = acc * dinv
    for c in range(4):
        out_ref[c] = hpn[:, c * 128:(c + 1) * 128]


def _mm_mid(agg, hp, dinv, b, w):
    return pl.pallas_call(
        _mm_mid_body,
        grid=(_N_TILES,),
        in_specs=[
            pl.BlockSpec((4, _ROW_BLK, 128), lambda i: (0, i, 0)),
            pl.BlockSpec((4, _ROW_BLK, 128), lambda i: (0, i, 0)),
            pl.BlockSpec((_ROW_BLK, 1), lambda i: (i, 0)),
            pl.BlockSpec((1, _H), lambda i: (0, 0)),
            pl.BlockSpec((_H, _H), lambda i: (0, 0)),
        ],
        out_specs=pl.BlockSpec((4, _ROW_BLK, 128), lambda i: (0, i, 0)),
        out_shape=jax.ShapeDtypeStruct((4, _N_PAD, 128), jnp.float32),
    )(agg, hp, dinv, b, w)


def _final_body(agg_ref, hp_ref, dinv_ref, b_ref, out_ref):
    dinv = dinv_ref[...]
    for c in range(4):
        out_ref[:, pl.ds(c * 128, 128)] = (
            dinv * (agg_ref[c] + hp_ref[c]) + b_ref[0, c * 128:(c + 1) * 128])


def _final(agg, hp, dinv, b):
    return pl.pallas_call(
        _final_body,
        grid=(_N_TILES,),
        in_specs=[
            pl.BlockSpec((4, _ROW_BLK, 128), lambda i: (0, i, 0)),
            pl.BlockSpec((4, _ROW_BLK, 128), lambda i: (0, i, 0)),
            pl.BlockSpec((_ROW_BLK, 1), lambda i: (i, 0)),
            pl.BlockSpec((1, _H), lambda i: (0, 0)),
        ],
        out_specs=pl.BlockSpec((_ROW_BLK, _H), lambda i: (i, 0)),
        out_shape=jax.ShapeDtypeStruct((_N_PAD, _H), jnp.float32),
    )(agg, hp, dinv, b)


def kernel(x, edge_index, W1, b1, W2, b2, W3, b3):
    npad = _E_PAD - _E
    src_p = jnp.concatenate(
        [edge_index[0], jnp.zeros((npad,), dtype=jnp.int32)])
    dst_p = jnp.concatenate(
        [edge_index[1], jnp.full((npad,), _N, dtype=jnp.int32)])
    src3 = src_p.reshape(_NT, _STEPS, 128)
    dst3 = dst_p.reshape(_NT, _STEPS, 128)
    dst3_deg = dst_p.reshape(_NSC * _NT, _DEG_STEPS, 128)
    x_pad = jnp.pad(x, ((0, _N_PAD - _N), (0, 0)))

    degf = _deg_sc(dst3_deg).reshape(_NSC, _N_PAD, 128)
    hp1, dinv = _mm_first(x_pad, degf, W1)
    agg1 = _scatter_sc(hp1.reshape(4 * _N_PAD, 128), src3, dst3)
    hp2 = _mm_mid(agg1.reshape(4, _N_PAD, 128), hp1, dinv,
                  b1.reshape(1, _H), W2)
    agg2 = _scatter_sc(hp2.reshape(4 * _N_PAD, 128), src3, dst3)
    hp3 = _mm_mid(agg2.reshape(4, _N_PAD, 128), hp2, dinv,
                  b2.reshape(1, _H), W3)
    agg3 = _scatter_sc(hp3.reshape(4 * _N_PAD, 128), src3, dst3)
    z = _final(agg3.reshape(4, _N_PAD, 128), hp3, dinv, b3.reshape(1, _H))
    return z[:_N]


# speedup vs baseline: 5.8444x; 1.1838x over previous
import functools

import jax
import jax.numpy as jnp
from jax import lax
from jax.experimental import pallas as pl
from jax.experimental.pallas import tpu as pltpu
from jax.experimental.pallas import tpu_sc as plsc

_N = 10000
_E = 160000
_F_IN = 256
_H = 512

_N_PAD = 10240
_ROW_BLK = 512
_N_TILES = _N_PAD // _ROW_BLK
_E_PAD = 163840
_STEPS = 80
_HSTEPS = 40
_DEG_STEPS = 40
_NSC = 2
_NT = 16
_STRIPE = _N_PAD // _NT


def _fill_2d(ref, rows, value):
    vec = jnp.full((16,), value, dtype=jnp.float32)

    def body(t, _):
        r = t // 8
        q = t % 8
        ref[r, pl.ds(q * 16, 16)] = vec
        return 0

    lax.fori_loop(0, rows * 8, body, 0)


def _deg_sc(dst3):
    mesh = plsc.VectorSubcoreMesh(core_axis_name="c", subcore_axis_name="s", num_cores=_NSC, num_subcores=_NT)

    @functools.partial(
        pl.kernel,
        out_type=jax.ShapeDtypeStruct((_NSC * _N_PAD, 128), jnp.float32),
        mesh=mesh,
        scratch_types=[
            pltpu.VMEM((_DEG_STEPS, 128), jnp.int32),
            pltpu.VMEM((128, 128), jnp.float32),
            pltpu.VMEM_SHARED((_N_PAD, 128), jnp.float32),
        ],
    )
    def body(dst_hbm, out_hbm, dst_v, ones_v, acc_sh):
        cid = lax.axis_index("c")
        sid = lax.axis_index("s")
        wid = cid * _NT + sid
        pltpu.sync_copy(dst_hbm.at[wid], dst_v)
        _fill_2d(ones_v, 128, 0.0)
        for k in range(_STRIPE // 128):
            pltpu.sync_copy(ones_v, acc_sh.at[pl.ds(sid * _STRIPE + k * 128, 128)])
        _fill_2d(ones_v, 128, 1.0)
        plsc.subcore_barrier()

        def step(j, _):
            pltpu.sync_copy(ones_v, acc_sh.at[dst_v.at[j]], add=True)
            return 0

        lax.fori_loop(0, _DEG_STEPS, step, 0)
        plsc.subcore_barrier()
        pltpu.sync_copy(
            acc_sh.at[pl.ds(sid * _STRIPE, _STRIPE)],
            out_hbm.at[pl.ds(cid * _N_PAD + sid * _STRIPE, _STRIPE)],
        )

    return body(dst3)


def _scatter_sc(hp_flat, src4, dst4):
    mesh = plsc.VectorSubcoreMesh(core_axis_name="c", subcore_axis_name="s", num_cores=_NSC, num_subcores=_NT)

    @functools.partial(
        pl.kernel,
        out_type=jax.ShapeDtypeStruct((4 * _N_PAD, 128), jnp.float32),
        mesh=mesh,
        scratch_types=[
            pltpu.VMEM((_HSTEPS, 128), jnp.int32),
            pltpu.VMEM((_HSTEPS, 128), jnp.int32),
            pltpu.VMEM((128, 128), jnp.float32),
            pltpu.VMEM((128, 128), jnp.float32),
            pltpu.VMEM_SHARED((_N_PAD, 128), jnp.float32),
            pltpu.SemaphoreType.DMA,
            pltpu.SemaphoreType.DMA,
        ],
    )
    def body(hp_hbm, src_hbm, dst_hbm, out_hbm,
             src_v, dst_v, rows0, rows1, acc_sh, sem0, sem1):
        cid = lax.axis_index("c")
        sid = lax.axis_index("s")
        _fill_2d(rows0, 128, 0.0)
        for k in range(_STRIPE // 128):
            pltpu.sync_copy(rows0, acc_sh.at[pl.ds(sid * _STRIPE + k * 128, 128)])

        def add_off(t, _, off_val=0):
            r = t // 8
            q = t % 8
            offv = jnp.full((16,), off_val, dtype=jnp.int32)
            src_v[r, pl.ds(q * 16, 16)] = src_v[r, pl.ds(q * 16, 16)] + offv
            return 0

        def wait_buf(buf, sem):
            pltpu.make_async_copy(hp_hbm.at[pl.ds(0, 128)], buf, sem).wait()

        for cc in range(2):
            off = (cid * 2 + cc) * _N_PAD
            plsc.subcore_barrier()
            for hh in range(2):
                w = sid * 2 + hh
                pltpu.sync_copy(src_hbm.at[w], src_v)
                pltpu.sync_copy(dst_hbm.at[w], dst_v)
                lax.fori_loop(0, _HSTEPS * 8,
                              functools.partial(add_off, off_val=off), 0)
                pltpu.async_copy(hp_hbm.at[src_v.at[0]], rows0, sem0)

                def pipe(jj, _):
                    j0 = jj * 2
                    j1 = jj * 2 + 1
                    pltpu.async_copy(hp_hbm.at[src_v.at[j1]], rows1, sem1)
                    wait_buf(rows0, sem0)
                    pltpu.sync_copy(rows0, acc_sh.at[dst_v.at[j0]], add=True)
                    nxt = lax.rem(j0 + 2, _HSTEPS)
                    pltpu.async_copy(hp_hbm.at[src_v.at[nxt]], rows0, sem0)
                    wait_buf(rows1, sem1)
                    pltpu.sync_copy(rows1, acc_sh.at[dst_v.at[j1]], add=True)
                    return 0

                lax.fori_loop(0, _HSTEPS // 2, pipe, 0)
                wait_buf(rows0, sem0)
            plsc.subcore_barrier()
            pltpu.sync_copy(
                acc_sh.at[pl.ds(sid * _STRIPE, _STRIPE)],
                out_hbm.at[pl.ds(off + sid * _STRIPE, _STRIPE)],
            )
            if cc == 0:
                _fill_2d(rows0, 128, 0.0)
                for k in range(_STRIPE // 128):
                    pltpu.sync_copy(
                        rows0, acc_sh.at[pl.ds(sid * _STRIPE + k * 128, 128)])

    return body(hp_flat, src4, dst4)


def _mm_first_body(x_ref, degf_ref, w_ref, hp_ref, dinv_ref):
    deg = degf_ref[0, :, 0:1] + degf_ref[1, :, 0:1] + 1.0
    dinv = lax.rsqrt(deg)
    h = jnp.dot(x_ref[...], w_ref[...], preferred_element_type=jnp.float32)
    hp = h * dinv
    for c in range(4):
        hp_ref[c] = hp[:, c * 128:(c + 1) * 128]
    dinv_ref[...] = dinv


def _mm_first(x_pad, degf, w1):
    return pl.pallas_call(
        _mm_first_body,
        grid=(_N_TILES,),
        in_specs=[
            pl.BlockSpec((_ROW_BLK, _F_IN), lambda i: (i, 0)),
            pl.BlockSpec((2, _ROW_BLK, 128), lambda i: (0, i, 0)),
            pl.BlockSpec((_F_IN, _H), lambda i: (0, 0)),
        ],
        out_specs=[
            pl.BlockSpec((4, _ROW_BLK, 128), lambda i: (0, i, 0)),
            pl.BlockSpec((_ROW_BLK, 1), lambda i: (i, 0)),
        ],
        out_shape=[
            jax.ShapeDtypeStruct((4, _N_PAD, 128), jnp.float32),
            jax.ShapeDtypeStruct((_N_PAD, 1), jnp.float32),
        ],
    )(x_pad, degf, w1)


def _mm_mid_body(agg_ref, hp_ref, dinv_ref, b_ref, w_ref, out_ref):
    dinv = dinv_ref[...]
    acc = jnp.zeros((_ROW_BLK, _H), dtype=jnp.float32)
    for c in range(4):
        zc = dinv * (agg_ref[c] + hp_ref[c]) + b_ref[0, c * 128:(c + 1) * 128]
        acc = acc + jnp.dot(zc, w_ref[pl.ds(c * 128, 128), :],
                            preferred_element_type=jnp.float32)
    hpn = acc * dinv
    for c in range(4):
        out_ref[c] = hpn[:, c * 128:(c + 1) * 128]


def _mm_mid(agg, hp, dinv, b, w):
    return pl.pallas_call(
        _mm_mid_body,
        grid=(_N_TILES,),
        in_specs=[
            pl.BlockSpec((4, _ROW_BLK, 128), lambda i: (0, i, 0)),
            pl.BlockSpec((4, _ROW_BLK, 128), lambda i: (0, i, 0)),
            pl.BlockSpec((_ROW_BLK, 1), lambda i: (i, 0)),
            pl.BlockSpec((1, _H), lambda i: (0, 0)),
            pl.BlockSpec((_H, _H), lambda i: (0, 0)),
        ],
        out_specs=pl.BlockSpec((4, _ROW_BLK, 128), lambda i: (0, i, 0)),
        out_shape=jax.ShapeDtypeStruct((4, _N_PAD, 128), jnp.float32),
    )(agg, hp, dinv, b, w)


def _final_body(agg_ref, hp_ref, dinv_ref, b_ref, out_ref):
    dinv = dinv_ref[...]
    for c in range(4):
        out_ref[:, pl.ds(c * 128, 128)] = (
            dinv * (agg_ref[c] + hp_ref[c]) + b_ref[0, c * 128:(c + 1) * 128])


def _final(agg, hp, dinv, b):
    return pl.pallas_call(
        _final_body,
        grid=(_N_TILES,),
        in_specs=[
            pl.BlockSpec((4, _ROW_BLK, 128), lambda i: (0, i, 0)),
            pl.BlockSpec((4, _ROW_BLK, 128), lambda i: (0, i, 0)),
            pl.BlockSpec((_ROW_BLK, 1), lambda i: (i, 0)),
            pl.BlockSpec((1, _H), lambda i: (0, 0)),
        ],
        out_specs=pl.BlockSpec((_ROW_BLK, _H), lambda i: (i, 0)),
        out_shape=jax.ShapeDtypeStruct((_N_PAD, _H), jnp.float32),
    )(agg, hp, dinv, b)


def kernel(x, edge_index, W1, b1, W2, b2, W3, b3):
    npad = _E_PAD - _E
    src_p = jnp.concatenate(
        [edge_index[0], jnp.zeros((npad,), dtype=jnp.int32)])
    dst_p = jnp.concatenate(
        [edge_index[1], jnp.full((npad,), _N, dtype=jnp.int32)])
    src3 = src_p.reshape(_NT * 2, _HSTEPS, 128)
    dst3 = dst_p.reshape(_NT * 2, _HSTEPS, 128)
    dst3_deg = dst_p.reshape(_NSC * _NT, _DEG_STEPS, 128)
    x_pad = jnp.pad(x, ((0, _N_PAD - _N), (0, 0)))

    degf = _deg_sc(dst3_deg).reshape(_NSC, _N_PAD, 128)
    hp1, dinv = _mm_first(x_pad, degf, W1)
    agg1 = _scatter_sc(hp1.reshape(4 * _N_PAD, 128), src3, dst3)
    hp2 = _mm_mid(agg1.reshape(4, _N_PAD, 128), hp1, dinv,
                  b1.reshape(1, _H), W2)
    agg2 = _scatter_sc(hp2.reshape(4 * _N_PAD, 128), src3, dst3)
    hp3 = _mm_mid(agg2.reshape(4, _N_PAD, 128), hp2, dinv,
                  b2.reshape(1, _H), W3)
    agg3 = _scatter_sc(hp3.reshape(4 * _N_PAD, 128), src3, dst3)
    z = _final(agg3.reshape(4, _N_PAD, 128), hp3, dinv, b3.reshape(1, _H))
    return z[:_N]
